# Initial kernel scaffold; baseline (speedup 1.0000x reference)
import functools

import jax
import jax.numpy as jnp
from jax import lax
from jax.experimental import pallas as pl
from jax.experimental.pallas import tpu as pltpu
from jax.experimental.pallas import tpu_sc as plsc

N_ST = 50000
N_T = 200
K = 8

A_SP = 0.15
A_PH = 0.05
A_CO = 0.1

NW = 32
S_PER_W = 1568
NP = NW * S_PER_W
PAD = NP - N_ST
CHUNKS = 7
C_ST = S_PER_W // CHUNKS
C_ROWS = C_ST * K
SUB = C_ROWS // 128
ROWS_W = S_PER_W * K // 128

LANE_PAD = 50048 - N_ST


def _tables_body(ph_ref, at_ref, ss_ref, sc_ref, cc_ref, m2_ref, phys_ref):
    p = ph_ref[...]
    s = jnp.sin(p)
    c = jnp.cos(p)
    ss_ref[...] = s * s
    sc_ref[...] = s * c
    cc_ref[...] = c * c
    m2_ref[...] = -2.0 * (s * c)
    a = at_ref[...]
    tot = jnp.sum(a, axis=0, keepdims=True)
    amp_pen = jnp.sum(jnp.maximum(tot - 80.0, 0.0))
    ann = a[2:3, :]
    ann_pen = jnp.sum(jnp.maximum(5.0 - ann, 0.0)) - 5.0 * LANE_PAD
    phys_ref[0, 0] = amp_pen / N_ST + 0.1 * (ann_pen / N_ST)


def _build_tables(ph_packed, amps_t):
    nrow = NP * 4 // 128
    plane = jax.ShapeDtypeStruct((nrow, 128), jnp.float32)
    return pl.pallas_call(
        _tables_body,
        out_shape=[plane, plane, plane, plane,
                   jax.ShapeDtypeStruct((1, 1), jnp.float32)],
    )(ph_packed, amps_t)


def _sc_spatial(f_tab, l_tab3, ni2, w2):
    mesh = plsc.VectorSubcoreMesh(core_axis_name="c", subcore_axis_name="s")

    @functools.partial(
        pl.kernel,
        mesh=mesh,
        out_type=jax.ShapeDtypeStruct((NW, 16), jnp.float32),
        scratch_types=[
            pltpu.VMEM((SUB, 128), jnp.int32),
            pltpu.VMEM((SUB, 128), jnp.float32),
            pltpu.VMEM((SUB, 16, 16), jnp.float32),
            pltpu.VMEM((SUB, 128, 16), jnp.float32),
            pltpu.VMEM((16,), jnp.float32),
            pltpu.SemaphoreType.DMA,
        ],
    )
    def body(f_hbm, l_hbm, ni_hbm, w_hbm, out_hbm,
             idx_v, w_v, l_v, rows_v, acc_v, sem):
        cid = lax.axis_index("c")
        sid = lax.axis_index("s")
        wid = sid * 2 + cid
        base_row = wid * ROWS_W
        base_grp = wid * (S_PER_W // 16)

        lane = lax.iota(jnp.int32, 16)
        amp_mask = lane < 4

        acc = jnp.zeros((16,), jnp.float32)
        for ch in range(CHUNKS):
            r0 = base_row + ch * SUB
            pltpu.sync_copy(ni_hbm.at[pl.ds(r0, SUB), :], idx_v)
            pltpu.sync_copy(w_hbm.at[pl.ds(r0, SUB), :], w_v)
            pltpu.sync_copy(l_hbm.at[pl.ds(base_grp + ch * SUB, SUB)], l_v)
            copies = [
                pltpu.make_async_copy(f_hbm.at[idx_v.at[j]], rows_v.at[j], sem)
                for j in range(SUB)
            ]
            for cp in copies:
                cp.start()
            for cp in copies:
                cp.wait()

            def row_body(r, acc):
                for g in range(16):
                    b = g * 8
                    gsum = w_v[r, b + 0] * rows_v[r, b + 0, :]
                    for k in range(1, 8):
                        gsum = gsum + w_v[r, b + k] * rows_v[r, b + k, :]
                    lrow = l_v[r, g, :]
                    d = lrow - gsum
                    acc = acc + jnp.where(amp_mask, d * d, lrow * gsum)
                return acc

            acc = lax.fori_loop(0, SUB, row_body, acc)

        acc_v[...] = acc
        pltpu.sync_copy(acc_v, out_hbm.at[wid])

    return body(f_tab, l_tab3, ni2, w2)


def _dense_body(p_ref, t_ref, phys_ref, sp_ref, out_ref):
    i = pl.program_id(0)
    p = p_ref[...]
    t = t_ref[...]
    d = p - t
    sd2 = jnp.sum(d * d)
    sp = jnp.sum(p, axis=1, keepdims=True)
    st = jnp.sum(t, axis=1, keepdims=True)
    spt = jnp.sum(p * t, axis=1, keepdims=True)
    spp = jnp.sum(p * p, axis=1, keepdims=True)
    stt = jnp.sum(t * t, axis=1, keepdims=True)
    inv_t = 1.0 / N_T
    num = spt - sp * st * inv_t
    vp = spp - sp * sp * inv_t
    vt = stt - st * st * inv_t
    corr = num / (jnp.sqrt(vp) * jnp.sqrt(vt) + 1e-8)
    csum = jnp.sum(corr)

    @pl.when(i == 0)
    def _():
        out_ref[0, 0] = (A_CO
                         + A_SP * jnp.sum(sp_ref[...]) / (N_ST * 8)
                         + A_PH * phys_ref[0, 0])

    out_ref[0, 0] += sd2 / (N_ST * N_T) - A_CO * csum / N_ST


def _dense(predicted, target, phys, sc_part):
    blk = 2000
    grid = N_ST // blk
    return pl.pallas_call(
        _dense_body,
        grid=(grid,),
        in_specs=[
            pl.BlockSpec((blk, N_T), lambda i: (i, 0)),
            pl.BlockSpec((blk, N_T), lambda i: (i, 0)),
            pl.BlockSpec((1, 1), lambda i: (0, 0)),
            pl.BlockSpec((NW, 16), lambda i: (0, 0)),
        ],
        out_specs=pl.BlockSpec((1, 1), lambda i: (0, 0)),
        out_shape=jax.ShapeDtypeStruct((1, 1), jnp.float32),
    )(predicted, target, phys, sc_part)


def kernel(predicted, target, seasonal_amplitudes, seasonal_phases,
           neighbor_weights, neighbor_indices):
    amps_p = jnp.pad(seasonal_amplitudes, ((0, PAD), (0, 0)))
    phases_p = jnp.pad(seasonal_phases, ((0, PAD), (0, 0)))
    ph_packed = phases_p.reshape(NP * 4 // 128, 128)
    amps_t = jnp.pad(seasonal_amplitudes.T, ((0, 0), (0, LANE_PAD)))

    ss, sc, cc, m2, phys = _build_tables(ph_packed, amps_t)
    ss4 = ss.reshape(NP, 4)
    sc4 = sc.reshape(NP, 4)
    cc4 = cc.reshape(NP, 4)
    m24 = m2.reshape(NP, 4)
    f_tab = jnp.concatenate([amps_p, ss4, sc4, cc4], axis=1)
    l_tab = jnp.concatenate([amps_p, cc4, m24, ss4], axis=1)
    l_tab3 = l_tab.reshape(NP // 16, 16, 16)

    ni2 = jnp.pad(neighbor_indices.astype(jnp.int32),
                  ((0, PAD), (0, 0))).reshape(NP * K // 128, 128)
    w2 = jnp.pad(neighbor_weights,
                 ((0, PAD), (0, 0))).reshape(NP * K // 128, 128)

    sc_part = _sc_spatial(f_tab, l_tab3, ni2, w2)

    out = _dense(predicted, target, phys, sc_part)
    return out[0, 0]


# speedup vs baseline: 3.3929x; 3.3929x over previous
import functools

import jax
import jax.numpy as jnp
from jax import lax
from jax.experimental import pallas as pl
from jax.experimental.pallas import tpu as pltpu
from jax.experimental.pallas import tpu_sc as plsc

N_ST = 50000
N_T = 200
K = 8

A_SP = 0.15
A_PH = 0.05
A_CO = 0.1

NW = 32
CHUNKS = 13
C_ST = 128
S_PER_W = CHUNKS * C_ST
NP = NW * S_PER_W
PAD = NP - N_ST
C_ROWS = C_ST * K
SUB = C_ROWS // 128
ROWS_W = S_PER_W * K // 128
LROW_W = S_PER_W * 16 // 256

LANE_PAD = 50048 - N_ST


def _tables_body(ph_ref, at_ref, ss_ref, sc_ref, cc_ref, m2_ref, phys_ref):
    p = ph_ref[...]
    s = jnp.sin(p)
    c = jnp.cos(p)
    ss_ref[...] = s * s
    sc_ref[...] = s * c
    cc_ref[...] = c * c
    m2_ref[...] = -2.0 * (s * c)
    a = at_ref[...]
    tot = jnp.sum(a, axis=0, keepdims=True)
    amp_pen = jnp.sum(jnp.maximum(tot - 80.0, 0.0))
    ann = a[2:3, :]
    ann_pen = jnp.sum(jnp.maximum(5.0 - ann, 0.0)) - 5.0 * LANE_PAD
    phys_ref[...] = jnp.reshape(amp_pen / N_ST + 0.1 * (ann_pen / N_ST), (1, 1))


def _build_tables(ph_packed, amps_t):
    nrow = NP * 4 // 128
    plane = jax.ShapeDtypeStruct((nrow, 128), jnp.float32)
    return pl.pallas_call(
        _tables_body,
        out_shape=[plane, plane, plane, plane,
                   jax.ShapeDtypeStruct((1, 1), jnp.float32)],
    )(ph_packed, amps_t)


def _sc_spatial(f_tab, l_tab3, ni2, w2):
    mesh = plsc.VectorSubcoreMesh(core_axis_name="c", subcore_axis_name="s",
                                  num_cores=2, num_subcores=16)

    @functools.partial(
        pl.kernel,
        mesh=mesh,
        out_type=jax.ShapeDtypeStruct((NW, 16), jnp.float32),
        scratch_types=[
            pltpu.VMEM((SUB, 128), jnp.int32),
            pltpu.VMEM((SUB, 128), jnp.float32),
            pltpu.VMEM((SUB, 256), jnp.float32),
            pltpu.VMEM((SUB, 128, 16), jnp.float32),
            pltpu.VMEM((16,), jnp.float32),
            pltpu.SemaphoreType.DMA,
        ],
        compiler_params=pltpu.CompilerParams(use_tc_tiling_on_sc=False),
    )
    def body(f_hbm, l_hbm, ni_hbm, w_hbm, out_hbm,
             idx_v, w_v, l_v, rows_v, acc_v, sem):
        cid = lax.axis_index("c")
        sid = lax.axis_index("s")
        wid = sid * 2 + cid
        base_row = wid * ROWS_W
        base_lrow = wid * LROW_W

        lane = lax.iota(jnp.int32, 16)
        amp_mask = lane < 4

        def chunk_body(ch, acc):
            r0 = base_row + ch * SUB
            pltpu.sync_copy(ni_hbm.at[pl.ds(r0, SUB), :], idx_v)
            pltpu.sync_copy(w_hbm.at[pl.ds(r0, SUB), :], w_v)
            pltpu.sync_copy(l_hbm.at[pl.ds(base_lrow + ch * SUB, SUB), :], l_v)
            copies = [
                pltpu.make_async_copy(f_hbm.at[idx_v.at[j]], rows_v.at[j], sem)
                for j in range(SUB)
            ]
            for cp in copies:
                cp.start()
            for cp in copies:
                cp.wait()

            def row_body(r, acc):
                wrow = [w_v[r, pl.ds(j * 16, 16)] for j in range(8)]
                for g in range(16):
                    b = g * 8
                    wv = wrow[g // 2]
                    off = (g % 2) * 8
                    gsum = wv[off] * rows_v[r, b, :]
                    for k in range(1, 8):
                        gsum = gsum + wv[off + k] * rows_v[r, b + k, :]
                    lrow = l_v[r, pl.ds(g * 16, 16)]
                    d = lrow - gsum
                    acc = acc + jnp.where(amp_mask, d * d, lrow * gsum)
                return acc

            return lax.fori_loop(0, SUB, row_body, acc)

        acc = lax.fori_loop(0, CHUNKS, chunk_body,
                            jnp.zeros((16,), jnp.float32))
        acc_v[...] = acc
        pltpu.sync_copy(acc_v, out_hbm.at[wid])

    return body(f_tab, l_tab3, ni2, w2)


def _dense_body(p_ref, t_ref, phys_ref, sp_ref, out_ref):
    i = pl.program_id(0)
    p = p_ref[...]
    t = t_ref[...]
    d = p - t
    sd2 = jnp.sum(d * d)
    sp = jnp.sum(p, axis=1, keepdims=True)
    st = jnp.sum(t, axis=1, keepdims=True)
    spt = jnp.sum(p * t, axis=1, keepdims=True)
    spp = jnp.sum(p * p, axis=1, keepdims=True)
    stt = jnp.sum(t * t, axis=1, keepdims=True)
    inv_t = 1.0 / N_T
    num = spt - sp * st * inv_t
    vp = spp - sp * sp * inv_t
    vt = stt - st * st * inv_t
    corr = num / (jnp.sqrt(vp) * jnp.sqrt(vt) + 1e-8)
    csum = jnp.sum(corr)

    @pl.when(i == 0)
    def _():
        out_ref[...] = jnp.reshape(
            A_CO + A_SP * jnp.sum(sp_ref[...]) / (N_ST * 8)
            + A_PH * phys_ref[0, 0], (1, 1))

    out_ref[...] = out_ref[...] + jnp.reshape(
        sd2 / (N_ST * N_T) - A_CO * csum / N_ST, (1, 1))


def _dense(predicted, target, phys, sc_part):
    blk = 2000
    grid = N_ST // blk
    return pl.pallas_call(
        _dense_body,
        grid=(grid,),
        in_specs=[
            pl.BlockSpec((blk, N_T), lambda i: (i, 0)),
            pl.BlockSpec((blk, N_T), lambda i: (i, 0)),
            pl.BlockSpec((1, 1), lambda i: (0, 0)),
            pl.BlockSpec((NW, 16), lambda i: (0, 0)),
        ],
        out_specs=pl.BlockSpec((1, 1), lambda i: (0, 0)),
        out_shape=jax.ShapeDtypeStruct((1, 1), jnp.float32),
    )(predicted, target, phys, sc_part)


def kernel(predicted, target, seasonal_amplitudes, seasonal_phases,
           neighbor_weights, neighbor_indices):
    amps_p = jnp.pad(seasonal_amplitudes, ((0, PAD), (0, 0)))
    phases_p = jnp.pad(seasonal_phases, ((0, PAD), (0, 0)))
    ph_packed = phases_p.reshape(NP * 4 // 128, 128)
    amps_t = jnp.pad(seasonal_amplitudes.T, ((0, 0), (0, LANE_PAD)))

    ss, sc, cc, m2, phys = _build_tables(ph_packed, amps_t)
    ss4 = ss.reshape(NP, 4)
    sc4 = sc.reshape(NP, 4)
    cc4 = cc.reshape(NP, 4)
    m24 = m2.reshape(NP, 4)
    f_tab = jnp.concatenate([amps_p, ss4, sc4, cc4], axis=1)
    l_tab = jnp.concatenate([amps_p, cc4, m24, ss4], axis=1)
    l_tab2 = l_tab.reshape(NP * 16 // 256, 256)

    ni2 = jnp.pad(neighbor_indices.astype(jnp.int32),
                  ((0, PAD), (0, 0))).reshape(NP * K // 128, 128)
    w2 = jnp.pad(neighbor_weights,
                 ((0, PAD), (0, 0))).reshape(NP * K // 128, 128)

    sc_part = _sc_spatial(f_tab, l_tab2, ni2, w2)

    out = _dense(predicted, target, phys, sc_part)
    return out[0, 0]


# speedup vs baseline: 5.9372x; 1.7499x over previous
import functools

import jax
import jax.numpy as jnp
import numpy as np
from jax import lax
from jax.experimental import pallas as pl
from jax.experimental.pallas import tpu as pltpu
from jax.experimental.pallas import tpu_sc as plsc

N_ST = 50000
N_T = 200
K = 8

A_SP = 0.15
A_PH = 0.05
A_CO = 0.1

NW = 32
CHUNKS = 13
C_ST = 128
S_PER_W = CHUNKS * C_ST
NP = NW * S_PER_W
PAD = NP - N_ST
SUB = C_ST * K // 128
ROWS_W = S_PER_W * K // 128

PACK = NP * 4 // 128
ROWS32 = NP // 8
PAD32 = ROWS32 - N_ST // 8


def _perm_matrix():
    p = np.zeros((128, 128), np.float32)
    for j in range(4):
        for g in range(8):
            for c in range(4):
                p[32 * j + 4 * g + c, 16 * g + 4 * j + c] = 1.0
    return p


def _sum4_matrix():
    s = np.zeros((32, 8), np.float32)
    for g in range(8):
        for c in range(4):
            s[4 * g + c, g] = 1.0
    return s


def _tables_body(ph_ref, am_ref, p_ref, s4_ref, f_ref, l_ref, phys_ref):
    ph = ph_ref[...]
    a = am_ref[...]
    s = jnp.sin(ph)
    c = jnp.cos(ph)
    ss = s * s
    sc = s * c
    cc = c * c
    perm = p_ref[...]
    f_ref[...] = jnp.dot(jnp.concatenate([a, ss, sc, cc], axis=1), perm,
                         preferred_element_type=jnp.float32)
    l_ref[...] = jnp.dot(jnp.concatenate([a, cc, -2.0 * sc, ss], axis=1), perm,
                         preferred_element_type=jnp.float32)
    tot = jnp.dot(a, s4_ref[...], preferred_element_type=jnp.float32)
    amp_pen = jnp.sum(jnp.maximum(tot - 80.0, 0.0))
    lane32 = lax.broadcasted_iota(jnp.int32, (1, 32), 1)
    is_ann = (lane32 % 4) == 2
    ann_pen = jnp.sum(jnp.where(is_ann, jnp.maximum(5.0 - a, 0.0), 0.0))
    ann_pen = ann_pen - 5.0 * PAD
    phys_ref[...] = jnp.reshape(amp_pen / N_ST + 0.1 * (ann_pen / N_ST), (1, 1))


def _build_tables(ph32, am32):
    plane = jax.ShapeDtypeStruct((ROWS32, 128), jnp.float32)
    return pl.pallas_call(
        _tables_body,
        out_shape=[plane, plane, jax.ShapeDtypeStruct((1, 1), jnp.float32)],
    )(ph32, am32, _perm_matrix(), _sum4_matrix())


def _sc_spatial(f_tab, l_tab2, ni2, w2):
    mesh = plsc.VectorSubcoreMesh(core_axis_name="c", subcore_axis_name="s",
                                  num_cores=2, num_subcores=16)

    @functools.partial(
        pl.kernel,
        mesh=mesh,
        out_type=jax.ShapeDtypeStruct((NW, 16), jnp.float32),
        scratch_types=[
            pltpu.VMEM((ROWS_W, 128), jnp.int32),
            pltpu.VMEM((ROWS_W, 128), jnp.float32),
            pltpu.VMEM((ROWS_W, 256), jnp.float32),
            pltpu.VMEM((SUB, 128, 16), jnp.float32),
            pltpu.VMEM((SUB, 128, 16), jnp.float32),
            pltpu.VMEM((16,), jnp.float32),
            pltpu.SemaphoreType.DMA,
            pltpu.SemaphoreType.DMA,
        ],
        compiler_params=pltpu.CompilerParams(use_tc_tiling_on_sc=False),
    )
    def body(f_hbm, l_hbm, ni_hbm, w_hbm, out_hbm,
             idx_v, w_v, l_v, rows_a, rows_b, acc_v, sem_a, sem_b):
        cid = lax.axis_index("c")
        sid = lax.axis_index("s")
        wid = sid * 2 + cid
        r0 = wid * ROWS_W

        lane = lax.iota(jnp.int32, 16)
        amp_mask = lane < 4

        pltpu.sync_copy(ni_hbm.at[pl.ds(r0, ROWS_W), :], idx_v)
        pltpu.sync_copy(w_hbm.at[pl.ds(r0, ROWS_W), :], w_v)
        pltpu.sync_copy(l_hbm.at[pl.ds(r0, ROWS_W), :], l_v)

        def fire(ch, buf, sem):
            for j in range(SUB):
                pltpu.make_async_copy(
                    f_hbm.at[idx_v.at[ch * SUB + j]], buf.at[j], sem).start()

        def drain(buf, sem):
            for j in range(SUB):
                pltpu.make_async_copy(
                    f_hbm.at[idx_v.at[j]], buf.at[j], sem).wait()

        def compute(ch, buf, acc):
            def row_body(r, acc):
                row = ch * SUB + r
                wrow = [w_v[row, pl.ds(j * 16, 16)] for j in range(8)]
                for g in range(16):
                    b = g * 8
                    wv = wrow[g // 2]
                    off = (g % 2) * 8
                    gsum = wv[off] * buf[r, b, :]
                    for k in range(1, 8):
                        gsum = gsum + wv[off + k] * buf[r, b + k, :]
                    lrow = l_v[row, pl.ds(g * 16, 16)]
                    d = lrow - gsum
                    acc = acc + jnp.where(amp_mask, d * d, lrow * gsum)
                return acc
            return lax.fori_loop(0, SUB, row_body, acc)

        fire(0, rows_a, sem_a)

        def pair_body(i, acc):
            ch0 = i * 2
            fire(ch0 + 1, rows_b, sem_b)
            drain(rows_a, sem_a)
            acc = compute(ch0, rows_a, acc)
            fire(ch0 + 2, rows_a, sem_a)
            drain(rows_b, sem_b)
            return compute(ch0 + 1, rows_b, acc)

        acc = lax.fori_loop(0, (CHUNKS - 1) // 2, pair_body,
                            jnp.zeros((16,), jnp.float32))
        drain(rows_a, sem_a)
        acc = compute(CHUNKS - 1, rows_a, acc)

        acc_v[...] = acc
        pltpu.sync_copy(acc_v, out_hbm.at[wid])

    return body(f_tab, l_tab2, ni2, w2)


def _dense_body(p_ref, t_ref, out_ref):
    i = pl.program_id(0)
    p = p_ref[...]
    t = t_ref[...]
    d = p - t
    sd2 = jnp.sum(d * d)
    sp = jnp.sum(p, axis=1, keepdims=True)
    st = jnp.sum(t, axis=1, keepdims=True)
    spt = jnp.sum(p * t, axis=1, keepdims=True)
    spp = jnp.sum(p * p, axis=1, keepdims=True)
    stt = jnp.sum(t * t, axis=1, keepdims=True)
    inv_t = 1.0 / N_T
    num = spt - sp * st * inv_t
    vp = spp - sp * sp * inv_t
    vt = stt - st * st * inv_t
    corr = num / (jnp.sqrt(vp) * jnp.sqrt(vt) + 1e-8)
    csum = jnp.sum(corr)

    @pl.when(i == 0)
    def _():
        out_ref[...] = jnp.zeros((1, 1), jnp.float32)

    out_ref[...] = out_ref[...] + jnp.reshape(
        sd2 / (N_ST * N_T) - A_CO * csum / N_ST, (1, 1))


def _dense(predicted, target):
    blk = 2000
    grid = N_ST // blk
    return pl.pallas_call(
        _dense_body,
        grid=(grid,),
        in_specs=[
            pl.BlockSpec((blk, N_T), lambda i: (i, 0)),
            pl.BlockSpec((blk, N_T), lambda i: (i, 0)),
        ],
        out_specs=pl.BlockSpec((1, 1), lambda i: (0, 0)),
        out_shape=jax.ShapeDtypeStruct((1, 1), jnp.float32),
    )(predicted, target)


def _combine_body(dn_ref, phys_ref, sp_ref, out_ref):
    out_ref[...] = jnp.reshape(
        dn_ref[0, 0] + A_CO
        + A_SP * jnp.sum(sp_ref[...]) / (N_ST * 8)
        + A_PH * phys_ref[0, 0], (1, 1))


def _combine(dense_part, phys, sc_part):
    return pl.pallas_call(
        _combine_body,
        out_shape=jax.ShapeDtypeStruct((1, 1), jnp.float32),
    )(dense_part, phys, sc_part)


def kernel(predicted, target, seasonal_amplitudes, seasonal_phases,
           neighbor_weights, neighbor_indices):
    am32 = jnp.pad(seasonal_amplitudes.reshape(N_ST // 8, 32),
                   ((0, PAD32), (0, 0)))
    ph32 = jnp.pad(seasonal_phases.reshape(N_ST // 8, 32),
                   ((0, PAD32), (0, 0)))

    f_packed, l_packed, phys = _build_tables(ph32, am32)
    f_tab = f_packed.reshape(NP, 16)
    l_tab2 = l_packed.reshape(NP * 16 // 256, 256)

    ni2 = jnp.pad(neighbor_indices.astype(jnp.int32).reshape(N_ST * K // 128, 128),
                  ((0, NP * K // 128 - N_ST * K // 128), (0, 0)))
    w2 = jnp.pad(neighbor_weights.reshape(N_ST * K // 128, 128),
                 ((0, NP * K // 128 - N_ST * K // 128), (0, 0)))

    sc_part = _sc_spatial(f_tab, l_tab2, ni2, w2)
    dense_part = _dense(predicted, target)
    out = _combine(dense_part, phys, sc_part)
    return out[0, 0]


# speedup vs baseline: 8.1029x; 1.3648x over previous
import functools

import jax
import jax.numpy as jnp
import numpy as np
from jax import lax
from jax.experimental import pallas as pl
from jax.experimental.pallas import tpu as pltpu
from jax.experimental.pallas import tpu_sc as plsc

N_ST = 50000
N_T = 200
K = 8

A_SP = 0.15
A_PH = 0.05
A_CO = 0.1

NW = 32
CHUNKS = 13
C_ST = 128
S_PER_W = CHUNKS * C_ST
NP = NW * S_PER_W
PAD = NP - N_ST
SUB = C_ST * K // 128
ROWS_W = S_PER_W * K // 128

ROWS32 = NP // 8
PAD32 = ROWS32 - N_ST // 8
NF = 50048
FR32 = NF // 8


def _perm_matrix():
    p = np.zeros((128, 128), np.float32)
    for j in range(4):
        for g in range(8):
            for c in range(4):
                p[32 * j + 4 * g + c, 16 * g + 4 * j + c] = 1.0
    return p


def _perm_matrix_split():
    p = np.zeros((128, 128), np.float32)
    for j in range(4):
        for g in range(8):
            for c in range(4):
                if c % 2 == 0:
                    tgt = 16 * g + 2 * j + c // 2
                else:
                    tgt = 16 * g + 8 + 2 * j + (c - 1) // 2
                p[32 * j + 4 * g + c, tgt] = 1.0
    return p


def _sum4_matrix():
    s = np.zeros((32, 8), np.float32)
    for g in range(8):
        for c in range(4):
            s[4 * g + c, g] = 1.0
    return s


def _tables_body(ph_ref, am_ref, p_ref, p2_ref, s4_ref, f_ref, l_ref, phys_ref):
    ph = ph_ref[...]
    a = am_ref[...]
    s = jnp.sin(ph)
    c = jnp.cos(ph)
    ss = s * s
    sc = s * c
    cc = c * c
    f_ref[...] = jnp.dot(
        jnp.concatenate([a[:FR32], ss[:FR32], sc[:FR32], cc[:FR32]], axis=1),
        p_ref[...], preferred_element_type=jnp.float32).astype(jnp.bfloat16)
    l_ref[...] = jnp.dot(jnp.concatenate([a, cc, -2.0 * sc, ss], axis=1),
                         p2_ref[...], preferred_element_type=jnp.float32)
    tot = jnp.dot(a, s4_ref[...], preferred_element_type=jnp.float32)
    amp_pen = jnp.sum(jnp.maximum(tot - 80.0, 0.0))
    lane32 = lax.broadcasted_iota(jnp.int32, (1, 32), 1)
    is_ann = (lane32 % 4) == 2
    ann_pen = jnp.sum(jnp.where(is_ann, jnp.maximum(5.0 - a, 0.0), 0.0))
    ann_pen = ann_pen - 5.0 * PAD
    phys_ref[...] = jnp.reshape(amp_pen / N_ST + 0.1 * (ann_pen / N_ST), (1, 1))


def _build_tables(ph32, am32):
    return pl.pallas_call(
        _tables_body,
        out_shape=[jax.ShapeDtypeStruct((FR32, 128), jnp.bfloat16),
                   jax.ShapeDtypeStruct((ROWS32, 128), jnp.float32),
                   jax.ShapeDtypeStruct((1, 1), jnp.float32)],
    )(ph32, am32, _perm_matrix(), _perm_matrix_split(), _sum4_matrix())


def _sc_spatial(f_tab, l_tab2, niw2):
    mesh = plsc.VectorSubcoreMesh(core_axis_name="c", subcore_axis_name="s",
                                  num_cores=2, num_subcores=16)

    @functools.partial(
        pl.kernel,
        mesh=mesh,
        out_type=jax.ShapeDtypeStruct((NW, 16), jnp.float32),
        scratch_types=[
            pltpu.VMEM((ROWS_W, 128), jnp.int32),
            pltpu.VMEM((ROWS_W, 128), jnp.int32),
            pltpu.VMEM((ROWS_W, 256), jnp.float32),
            pltpu.VMEM((SUB, 128, 8), jnp.int32),
            pltpu.VMEM((SUB, 128, 8), jnp.int32),
            pltpu.VMEM((16,), jnp.float32),
            pltpu.VMEM((32,), jnp.float32),
            pltpu.VMEM_SHARED((NF, 8), jnp.int32),
            pltpu.SemaphoreType.DMA,
            pltpu.SemaphoreType.DMA,
        ],
        compiler_params=pltpu.CompilerParams(use_tc_tiling_on_sc=False,
                                             needs_layout_passes=False),
    )
    def body(f_hbm, l_hbm, niw_hbm, out_hbm,
             idx_v, niw_v, l_v, rows_a, rows_b, acc_v, fold_v, f_sh,
             sem_a, sem_b):
        cid = lax.axis_index("c")
        sid = lax.axis_index("s")
        wid = sid * 2 + cid
        r0 = wid * ROWS_W

        lane = lax.iota(jnp.int32, 16)
        amp_mask = (lane & 7) < 2
        lo_mask = lane < 8
        half01 = (lane >= 8).astype(jnp.int32)
        lane7 = lane & 7
        fold_i1 = lane + 8 * half01
        fold_i2 = fold_i1 + 8
        wmask = jnp.full((16,), jnp.int32(-65536))

        frows = NF // 16
        pltpu.sync_copy(f_hbm.at[pl.ds(sid * frows, frows), :],
                        f_sh.at[pl.ds(sid * frows, frows), :])
        pltpu.sync_copy(niw_hbm.at[pl.ds(r0, ROWS_W), :], niw_v)
        pltpu.sync_copy(l_hbm.at[pl.ds(r0, ROWS_W), :], l_v)

        def dec_body(r, carry):
            for j in range(8):
                v = niw_v[r, pl.ds(j * 16, 16)]
                idx_v[r, pl.ds(j * 16, 16)] = v & 0xFFFF
            return carry
        lax.fori_loop(0, ROWS_W, dec_body, 0)
        plsc.subcore_barrier()

        def fire(ch, buf, sem):
            for j in range(SUB):
                pltpu.make_async_copy(
                    f_sh.at[idx_v.at[ch * SUB + j]], buf.at[j], sem).start()

        def drain(buf, sem):
            for j in range(SUB):
                pltpu.make_async_copy(
                    f_sh.at[idx_v.at[j]], buf.at[j], sem).wait()

        def compute(ch, buf, acc):
            def row_body(r, acc):
                row = ch * SUB + r
                r_vec = lane7 * 0 + r
                wrow = [
                    plsc.bitcast(niw_v[row, pl.ds(j * 16, 16)] & wmask,
                                 jnp.float32)
                    for j in range(8)
                ]
                for st in range(16):
                    wv = wrow[st // 2]
                    off = (st % 2) * 8
                    base = half01 + st * 8
                    acc_e = jnp.zeros((16,), jnp.float32)
                    acc_o = jnp.zeros((16,), jnp.float32)
                    for t in range(4):
                        v = plsc.load_gather(buf, [r_vec, base + 2 * t, lane7])
                        evens = plsc.bitcast(v << 16, jnp.float32)
                        odds = plsc.bitcast(v & wmask, jnp.float32)
                        wpair = jnp.where(lo_mask, wv[off + 2 * t],
                                          wv[off + 2 * t + 1])
                        acc_e = acc_e + wpair * evens
                        acc_o = acc_o + wpair * odds
                    fold_v[pl.ds(0, 16)] = acc_e
                    fold_v[pl.ds(16, 16)] = acc_o
                    gsum = (plsc.load_gather(fold_v, [fold_i1])
                            + plsc.load_gather(fold_v, [fold_i2]))
                    lrow = l_v[row, pl.ds(st * 16, 16)]
                    d = lrow - gsum
                    acc = acc + jnp.where(amp_mask, d * d, lrow * gsum)
                return acc
            return lax.fori_loop(0, SUB, row_body, acc)

        fire(0, rows_a, sem_a)

        def pair_body(i, acc):
            ch0 = i * 2
            fire(ch0 + 1, rows_b, sem_b)
            drain(rows_a, sem_a)
            acc = compute(ch0, rows_a, acc)
            fire(ch0 + 2, rows_a, sem_a)
            drain(rows_b, sem_b)
            return compute(ch0 + 1, rows_b, acc)

        acc = lax.fori_loop(0, (CHUNKS - 1) // 2, pair_body,
                            jnp.zeros((16,), jnp.float32))
        drain(rows_a, sem_a)
        acc = compute(CHUNKS - 1, rows_a, acc)

        acc_v[...] = acc
        pltpu.sync_copy(acc_v, out_hbm.at[wid])

    return body(f_tab, l_tab2, niw2)


def _dense_body(p_ref, t_ref, out_ref):
    i = pl.program_id(0)
    p = p_ref[...]
    t = t_ref[...]
    d = p - t
    sd2 = jnp.sum(d * d)
    sp = jnp.sum(p, axis=1, keepdims=True)
    st = jnp.sum(t, axis=1, keepdims=True)
    spt = jnp.sum(p * t, axis=1, keepdims=True)
    spp = jnp.sum(p * p, axis=1, keepdims=True)
    stt = jnp.sum(t * t, axis=1, keepdims=True)
    inv_t = 1.0 / N_T
    num = spt - sp * st * inv_t
    vp = spp - sp * sp * inv_t
    vt = stt - st * st * inv_t
    corr = num / (jnp.sqrt(vp) * jnp.sqrt(vt) + 1e-8)
    csum = jnp.sum(corr)

    @pl.when(i == 0)
    def _():
        out_ref[...] = jnp.zeros((1, 1), jnp.float32)

    out_ref[...] = out_ref[...] + jnp.reshape(
        sd2 / (N_ST * N_T) - A_CO * csum / N_ST, (1, 1))


def _dense(predicted, target):
    blk = 2000
    grid = N_ST // blk
    return pl.pallas_call(
        _dense_body,
        grid=(grid,),
        in_specs=[
            pl.BlockSpec((blk, N_T), lambda i: (i, 0)),
            pl.BlockSpec((blk, N_T), lambda i: (i, 0)),
        ],
        out_specs=pl.BlockSpec((1, 1), lambda i: (0, 0)),
        out_shape=jax.ShapeDtypeStruct((1, 1), jnp.float32),
    )(predicted, target)


def _combine_body(dn_ref, phys_ref, sp_ref, out_ref):
    out_ref[...] = jnp.reshape(
        dn_ref[0, 0] + A_CO
        + A_SP * jnp.sum(sp_ref[...]) / (N_ST * 8)
        + A_PH * phys_ref[0, 0], (1, 1))


def _combine(dense_part, phys, sc_part):
    return pl.pallas_call(
        _combine_body,
        out_shape=jax.ShapeDtypeStruct((1, 1), jnp.float32),
    )(dense_part, phys, sc_part)


def kernel(predicted, target, seasonal_amplitudes, seasonal_phases,
           neighbor_weights, neighbor_indices):
    am32 = jnp.pad(seasonal_amplitudes.reshape(N_ST // 8, 32),
                   ((0, PAD32), (0, 0)))
    ph32 = jnp.pad(seasonal_phases.reshape(N_ST // 8, 32),
                   ((0, PAD32), (0, 0)))

    f_packed, l_packed, phys = _build_tables(ph32, am32)
    f_tab = jax.lax.bitcast_convert_type(
        f_packed.reshape(FR32, 64, 2), jnp.int32).reshape(NF, 8)
    l_tab2 = l_packed.reshape(NP * 16 // 256, 256)

    wu = jax.lax.bitcast_convert_type(neighbor_weights, jnp.uint32)
    wu = wu + jnp.uint32(0x7FFF) + ((wu >> 16) & jnp.uint32(1))
    wbits = (wu & jnp.uint32(0xFFFF0000)).astype(jnp.int32)
    niw = neighbor_indices.astype(jnp.int32) | wbits
    niw2 = jnp.pad(niw.reshape(N_ST * K // 128, 128),
                   ((0, NP * K // 128 - N_ST * K // 128), (0, 0)))

    sc_part = _sc_spatial(f_tab, l_tab2, niw2)
    dense_part = _dense(predicted, target)
    out = _combine(dense_part, phys, sc_part)
    return out[0, 0]
